# baseline (speedup 1.0000x reference)
import functools

import jax
import jax.numpy as jnp
from jax import lax
from jax.experimental import pallas as pl
from jax.experimental.pallas import tpu as pltpu
from jax.experimental.pallas import tpu_sc as plsc

B, N, H = 16, 256, 16
NBOND = 20
NDIST = 512
E_TOTAL = B * N * N
PLANE = N * N
HW = H // 2


def _build_sc_kernel():
    info = plsc.get_sparse_core_info()
    NC, NS, L = info.num_cores, info.num_subcores, info.num_lanes
    NW = NC * NS
    per_w = E_TOTAL // NW
    CH = 1024
    CHR = CH // N
    n_chunks = per_w // CH
    halves = PLANE // per_w

    mesh = plsc.VectorSubcoreMesh(core_axis_name="c", subcore_axis_name="s")

    @functools.partial(
        pl.kernel,
        mesh=mesh,
        out_type=jax.ShapeDtypeStruct((B, H, N, N), jnp.float32),
        compiler_params=pltpu.CompilerParams(needs_layout_passes=False),
        scratch_types=[
            pltpu.VMEM((NBOND * H,), jnp.float32),
            pltpu.VMEM((NDIST * H,), jnp.float32),
            pltpu.VMEM((NBOND * (HW + 1),), jnp.int32),
            pltpu.VMEM((NDIST * (HW + 1),), jnp.int32),
            pltpu.VMEM((CHR, N), jnp.int32),
            pltpu.VMEM((CHR, N), jnp.int32),
            pltpu.VMEM((CHR, N), jnp.int32),
            pltpu.VMEM((CHR, N), jnp.int32),
            pltpu.VMEM((H, CHR, N), jnp.float32),
            pltpu.VMEM((H, CHR, N), jnp.float32),
            pltpu.SemaphoreType.DMA,
            pltpu.SemaphoreType.DMA,
            pltpu.SemaphoreType.DMA,
            pltpu.SemaphoreType.DMA,
        ],
    )
    def sc_kernel(adj_hbm, dist_hbm, et_hbm, dt_hbm, out_hbm,
                  et_v, dt_v, et8_v, dt8_v,
                  adj_v0, adj_v1, dist_v0, dist_v1,
                  stage0, stage1, si0, si1, so0, so1):
        wid = lax.axis_index("s") * NC + lax.axis_index("c")
        b = wid // halves
        half = wid % halves
        row_base = half * (per_w // N)
        adj_b = (adj_v0, adj_v1)
        dist_b = (dist_v0, dist_v1)
        stage_b = (stage0, stage1)
        sem_i = (si0, si1)
        sem_o = (so0, so1)

        def issue_idx(ci, s):
            gr = row_base + ci * CHR
            pltpu.async_copy(adj_hbm.at[b, pl.ds(gr, CHR), pl.ds(0, N)],
                             adj_b[s], sem_i[s])
            pltpu.async_copy(dist_hbm.at[b, pl.ds(gr, CHR), pl.ds(0, N)],
                             dist_b[s], sem_i[s])

        def wait_idx(s):
            pltpu.make_async_copy(adj_hbm.at[0, pl.ds(0, CHR), pl.ds(0, N)],
                                  adj_b[s], sem_i[s]).wait()
            pltpu.make_async_copy(dist_hbm.at[0, pl.ds(0, CHR), pl.ds(0, N)],
                                  dist_b[s], sem_i[s]).wait()

        def issue_out(ci, s):
            r0 = row_base + ci * CHR
            pltpu.async_copy(
                stage_b[s],
                out_hbm.at[b, pl.ds(0, H), pl.ds(r0, CHR), pl.ds(0, N)],
                sem_o[s])

        def wait_out(s):
            pltpu.make_async_copy(
                stage_b[s],
                out_hbm.at[0, pl.ds(0, H), pl.ds(0, CHR), pl.ds(0, N)],
                sem_o[s]).wait()

        issue_idx(0, 0)
        issue_idx(1, 1)
        pltpu.sync_copy(et_hbm, et_v)
        pltpu.sync_copy(dt_hbm, dt_v)

        lanes = lax.iota(jnp.int32, L)

        def pack_table(src_v, dst_v, n_words):
            @plsc.parallel_loop(0, n_words // L, unroll=2)
            def pack_body(j):
                widx = j * L + lanes
                lo = plsc.load_gather(src_v, [widx * 2])
                hi = plsc.load_gather(src_v, [widx * 2 + 1])
                pair = plsc.pack(lo, hi, format=plsc.PackFormat.INTERLEAVED)
                didx = (widx >> 3) * 9 + (widx & 7)
                plsc.store_scatter(dst_v, [didx], plsc.bitcast(pair, jnp.int32))

        pack_table(et_v, et8_v, NBOND * HW)
        pack_table(dt_v, dt8_v, NDIST * HW)

        MASK_HI = jnp.int32(-65536)
        BLK = N // L

        def compute(s):
            asv = adj_b[s]
            dsv = dist_b[s]
            stg = stage_b[s]

            @plsc.parallel_loop(0, CH // L, unroll=4)
            def vec_body(i):
                r = i // BLK
                c = (i % BLK) * L
                av = asv[r, pl.ds(c, L)]
                dv = dsv[r, pl.ds(c, L)]
                a8 = (av << 3) + av
                d8 = (dv << 3) + dv
                ewords = [plsc.load_gather(et8_v, [a8 + w]) for w in range(HW)]
                dwords = [plsc.load_gather(dt8_v, [d8 + w]) for w in range(HW)]
                for w in range(HW):
                    ew, dw = ewords[w], dwords[w]
                    stg[2 * w, r, pl.ds(c, L)] = (
                        plsc.bitcast(ew << 16, jnp.float32)
                        + plsc.bitcast(dw << 16, jnp.float32))
                    stg[2 * w + 1, r, pl.ds(c, L)] = (
                        plsc.bitcast(ew & MASK_HI, jnp.float32)
                        + plsc.bitcast(dw & MASK_HI, jnp.float32))

        for s in (0, 1):
            wait_idx(s)
            compute(s)
            issue_out(s, s)
            issue_idx(s + 2, s)

        def pair_body(p, carry):
            for s in (0, 1):
                ci = 2 * p + s
                wait_idx(s)
                wait_out(s)
                compute(s)
                issue_out(ci, s)
                issue_idx(ci + 2, s)
            return carry

        lax.fori_loop(1, n_chunks // 2 - 1, pair_body, 0, unroll=False)

        for s in (0, 1):
            ci = n_chunks - 2 + s
            wait_idx(s)
            wait_out(s)
            compute(s)
            issue_out(ci, s)
        for s in (0, 1):
            wait_out(s)

    return sc_kernel


def kernel(adj, distance, edge_table, dist_table):
    sc = _build_sc_kernel()
    return sc(adj, distance,
              edge_table.reshape(-1), dist_table.reshape(-1))


# speedup vs baseline: 2.2116x; 2.2116x over previous
import functools

import jax
import jax.numpy as jnp
from jax import lax
from jax.experimental import pallas as pl
from jax.experimental.pallas import tpu as pltpu
from jax.experimental.pallas import tpu_sc as plsc

B, N, H = 16, 256, 16
NBOND = 20
NDIST = 512
E_TOTAL = B * N * N
PLANE = N * N
HW = H // 2


def _build_sc_kernel():
    info = plsc.get_sparse_core_info()
    NC, NS, L = info.num_cores, info.num_subcores, info.num_lanes
    NW = NC * NS
    per_w = E_TOTAL // NW
    CH = 2048
    CHR = CH // N
    n_chunks = per_w // CH
    halves = PLANE // per_w

    mesh = plsc.VectorSubcoreMesh(core_axis_name="c", subcore_axis_name="s")

    @functools.partial(
        pl.kernel,
        mesh=mesh,
        out_type=jax.ShapeDtypeStruct((B, H, N, N), jnp.float32),
        compiler_params=pltpu.CompilerParams(needs_layout_passes=False),
        scratch_types=[
            pltpu.VMEM((NBOND * H,), jnp.float32),
            pltpu.VMEM((NDIST * H,), jnp.float32),
            pltpu.VMEM((NBOND * (HW + 1),), jnp.int32),
            pltpu.VMEM((NDIST * (HW + 1),), jnp.int32),
            pltpu.VMEM((CHR, N), jnp.int32),
            pltpu.VMEM((CHR, N), jnp.int32),
            pltpu.VMEM((CHR, N), jnp.int32),
            pltpu.VMEM((CHR, N), jnp.int32),
            pltpu.VMEM((H, CHR, N), jnp.float32),
            pltpu.VMEM((H, CHR, N), jnp.float32),
            pltpu.SemaphoreType.DMA,
            pltpu.SemaphoreType.DMA,
            pltpu.SemaphoreType.DMA,
            pltpu.SemaphoreType.DMA,
        ],
    )
    def sc_kernel(adj_hbm, dist_hbm, et_hbm, dt_hbm, out_hbm,
                  et_v, dt_v, et8_v, dt8_v,
                  adj_v0, adj_v1, dist_v0, dist_v1,
                  stage0, stage1, si0, si1, so0, so1):
        wid = lax.axis_index("s") * NC + lax.axis_index("c")
        b = wid // halves
        half = wid % halves
        row_base = half * (per_w // N)
        adj_b = (adj_v0, adj_v1)
        dist_b = (dist_v0, dist_v1)
        stage_b = (stage0, stage1)
        sem_i = (si0, si1)
        sem_o = (so0, so1)

        def issue_idx(ci, s):
            gr = row_base + ci * CHR
            pltpu.async_copy(adj_hbm.at[b, pl.ds(gr, CHR), pl.ds(0, N)],
                             adj_b[s], sem_i[s])
            pltpu.async_copy(dist_hbm.at[b, pl.ds(gr, CHR), pl.ds(0, N)],
                             dist_b[s], sem_i[s])

        def wait_idx(s):
            pltpu.make_async_copy(adj_hbm.at[0, pl.ds(0, CHR), pl.ds(0, N)],
                                  adj_b[s], sem_i[s]).wait()
            pltpu.make_async_copy(dist_hbm.at[0, pl.ds(0, CHR), pl.ds(0, N)],
                                  dist_b[s], sem_i[s]).wait()

        def issue_out(ci, s):
            r0 = row_base + ci * CHR
            pltpu.async_copy(
                stage_b[s],
                out_hbm.at[b, pl.ds(0, H), pl.ds(r0, CHR), pl.ds(0, N)],
                sem_o[s])

        def wait_out(s):
            pltpu.make_async_copy(
                stage_b[s],
                out_hbm.at[0, pl.ds(0, H), pl.ds(0, CHR), pl.ds(0, N)],
                sem_o[s]).wait()

        issue_idx(0, 0)
        issue_idx(1, 1)
        pltpu.sync_copy(et_hbm, et_v)
        pltpu.sync_copy(dt_hbm, dt_v)

        lanes = lax.iota(jnp.int32, L)

        def pack_table(src_v, dst_v, n_words):
            @plsc.parallel_loop(0, n_words // L, unroll=2)
            def pack_body(j):
                widx = j * L + lanes
                lo = plsc.load_gather(src_v, [widx * 2])
                hi = plsc.load_gather(src_v, [widx * 2 + 1])
                pair = plsc.pack(lo, hi, format=plsc.PackFormat.INTERLEAVED)
                didx = (widx >> 3) * 9 + (widx & 7)
                plsc.store_scatter(dst_v, [didx], plsc.bitcast(pair, jnp.int32))

        pack_table(et_v, et8_v, NBOND * HW)
        pack_table(dt_v, dt8_v, NDIST * HW)

        MASK_HI = jnp.int32(-65536)
        BLK = N // L

        def compute(s):
            asv = adj_b[s]
            dsv = dist_b[s]
            stg = stage_b[s]

            @plsc.parallel_loop(0, CH // L, unroll=2)
            def vec_body(i):
                r = i // BLK
                c = (i % BLK) * L
                av = asv[r, pl.ds(c, L)]
                dv = dsv[r, pl.ds(c, L)]
                a8 = (av << 3) + av
                d8 = (dv << 3) + dv
                ewords = [plsc.load_gather(et8_v, [a8 + w]) for w in range(HW)]
                dwords = [plsc.load_gather(dt8_v, [d8 + w]) for w in range(HW)]
                for w in range(HW):
                    ew, dw = ewords[w], dwords[w]
                    stg[2 * w, r, pl.ds(c, L)] = (
                        plsc.bitcast(ew << 16, jnp.float32)
                        + plsc.bitcast(dw << 16, jnp.float32))
                    stg[2 * w + 1, r, pl.ds(c, L)] = (
                        plsc.bitcast(ew & MASK_HI, jnp.float32)
                        + plsc.bitcast(dw & MASK_HI, jnp.float32))

        for s in (0, 1):
            wait_idx(s)
            compute(s)
            issue_out(s, s)
            issue_idx(s + 2, s)

        def pair_body(p, carry):
            for s in (0, 1):
                ci = 2 * p + s
                wait_idx(s)
                wait_out(s)
                compute(s)
                issue_out(ci, s)
                issue_idx(ci + 2, s)
            return carry

        lax.fori_loop(1, n_chunks // 2 - 1, pair_body, 0, unroll=False)

        for s in (0, 1):
            ci = n_chunks - 2 + s
            wait_idx(s)
            wait_out(s)
            compute(s)
            issue_out(ci, s)
        for s in (0, 1):
            wait_out(s)

    return sc_kernel


def kernel(adj, distance, edge_table, dist_table):
    sc = _build_sc_kernel()
    return sc(adj, distance,
              edge_table.reshape(-1), dist_table.reshape(-1))
